# baseline (speedup 1.0000x reference)
import functools

import jax
import jax.numpy as jnp
from jax import lax
from jax.experimental import pallas as pl
from jax.experimental.pallas import tpu as pltpu
from jax.experimental.pallas import tpu_sc as plsc

_NC = 2
_NS = 16
_NW = _NC * _NS
_L = 16
_C = 128
_K = 80
_EPT = _K * _C
_EP = _NW * _EPT
_NG = 4
_NB = 2 * _NG
_RPT = 640
_NPAD = _NS * _RPT


def _mm_body(x_ref, w_ref, o_ref):
    o_ref[...] = jnp.dot(x_ref[...], w_ref[...],
                         preferred_element_type=jnp.float32)


def _matmul(x, w, block_rows):
    n, din = x.shape
    dout = w.shape[1]
    return pl.pallas_call(
        _mm_body,
        grid=(n // block_rows,),
        in_specs=[
            pl.BlockSpec((block_rows, din), lambda i: (i, 0)),
            pl.BlockSpec((din, dout), lambda i: (0, 0)),
        ],
        out_specs=pl.BlockSpec((block_rows, dout), lambda i: (i, 0)),
        out_shape=jax.ShapeDtypeStruct((n, dout), jnp.float32),
    )(x, w)


def _combine_mm_body(p_ref, w_ref, o_ref):
    h = jnp.maximum(p_ref[0] + p_ref[1], 0.0)
    o_ref[...] = jnp.dot(h, w_ref[...], preferred_element_type=jnp.float32)


def _combine_relu_mm(p, w, block_rows=1024):
    n = p.shape[1]
    dout = w.shape[1]
    return pl.pallas_call(
        _combine_mm_body,
        grid=(n // block_rows,),
        in_specs=[
            pl.BlockSpec((2, block_rows, p.shape[2]), lambda i: (0, i, 0)),
            pl.BlockSpec(w.shape, lambda i: (0, 0)),
        ],
        out_specs=pl.BlockSpec((block_rows, dout), lambda i: (i, 0)),
        out_shape=jax.ShapeDtypeStruct((n, dout), jnp.float32),
    )(p, w)


def _combine_body(p_ref, o_ref):
    o_ref[...] = p_ref[0] + p_ref[1]


def _combine(p, block_rows=1024):
    n, d = p.shape[1], p.shape[2]
    return pl.pallas_call(
        _combine_body,
        grid=(n // block_rows,),
        in_specs=[pl.BlockSpec((2, block_rows, d), lambda i: (0, i, 0))],
        out_specs=pl.BlockSpec((block_rows, d), lambda i: (i, 0)),
        out_shape=jax.ShapeDtypeStruct((n, d), jnp.float32),
    )(p)


def _sc_aggregate(h, src3, dst3, w2):
    mesh = plsc.VectorSubcoreMesh(core_axis_name="c", subcore_axis_name="s")

    @functools.partial(
        pl.kernel,
        out_type=jax.ShapeDtypeStruct((_NC, _NPAD, 16), jnp.float32),
        mesh=mesh,
        scratch_types=[
            pltpu.VMEM((_K, _C), jnp.int32),
            pltpu.VMEM((_K, _C), jnp.int32),
            pltpu.VMEM((_K, _C), jnp.float32),
        ] + [pltpu.VMEM((_C, 16), jnp.float32) for _ in range(_NB)] + [
            pltpu.VMEM((_C, 16), jnp.float32),
            pltpu.VMEM_SHARED((_NPAD, 16), jnp.float32),
        ] + [pltpu.SemaphoreType.DMA for _ in range(_NB)],
        compiler_params=pltpu.CompilerParams(use_tc_tiling_on_sc=False),
    )
    def agg(h_hbm, src_hbm, dst_hbm, w_hbm, out_hbm, src_v, dst_v, w_v,
            *bufs):
        rows = bufs[:_NB]
        zbuf = bufs[_NB]
        acc = bufs[_NB + 1]
        sems = bufs[_NB + 2:]
        core = lax.axis_index("c")
        sub = lax.axis_index("s")
        tid = (1 - core) * _NS + sub

        pltpu.sync_copy(src_hbm.at[tid], src_v)
        pltpu.sync_copy(dst_hbm.at[tid], dst_v)
        pltpu.sync_copy(w_hbm.at[tid], w_v)

        zero = jnp.zeros((_L,), jnp.float32)
        for i in range(_C):
            zbuf[i] = zero
        for k in range(_RPT // _C):
            pltpu.sync_copy(zbuf, acc.at[pl.ds(sub * _RPT + k * _C, _C)])
        plsc.subcore_barrier()

        dnums = lax.GatherDimensionNumbers(
            offset_dims=(), collapsed_slice_dims=(0,), start_index_map=(0,))

        def gath(c, b):
            return pltpu.async_copy(h_hbm.at[src_v.at[c]], rows[b], sems[b])

        def proc(c, b):
            for g in range(_C // _L):
                w16 = w_v[c, pl.ds(g * _L, _L)]
                for e in range(_L):
                    wb = lax.gather(
                        w16, jnp.full((_L, 1), e, jnp.int32), dnums, (1,),
                        mode=lax.GatherScatterMode.PROMISE_IN_BOUNDS)
                    r = g * _L + e
                    rows[b][r] = rows[b][r] * wb
            return pltpu.async_copy(rows[b], acc.at[dst_v.at[c]], sems[b],
                                    add=True)

        primeA = [gath(b, b) for b in range(_NG)]

        def super_body(j, carry):
            base = j * 2 * _NG
            gB = [gath(base + _NG + b, _NG + b) for b in range(_NG)]
            sA = []
            for b in range(_NG):
                pltpu.make_async_copy(
                    h_hbm.at[src_v.at[base + b]], rows[b], sems[b]).wait()
                sA.append(proc(base + b, b))
            for b in range(_NG):
                sA[b].wait()

            @pl.when(j < _K // (2 * _NG) - 1)
            def _():
                for b in range(_NG):
                    gath(base + 2 * _NG + b, b)

            for b in range(_NG):
                gB[b].wait()
            sB = [proc(base + _NG + b, _NG + b) for b in range(_NG)]
            for b in range(_NG):
                sB[b].wait()
            return carry

        lax.fori_loop(0, _K // (2 * _NG), super_body, 0)
        plsc.subcore_barrier()
        pltpu.sync_copy(acc.at[pl.ds(sub * _RPT, _RPT)],
                        out_hbm.at[core, pl.ds(sub * _RPT, _RPT)])

    return agg(h, src3, dst3, w2)


def kernel(x, edge_index, edge_weight, W1, W2):
    n = x.shape[0]
    e = edge_index.shape[1]
    ei = edge_index.astype(jnp.int32)
    w = edge_weight.astype(jnp.float32)
    pad = _EP - e
    pad_dst = n + (jnp.arange(pad, dtype=jnp.int32) % (_NPAD - n))
    src = jnp.concatenate([ei[0], jnp.zeros((pad,), jnp.int32)])
    dst = jnp.concatenate([ei[1], pad_dst])
    wp = jnp.concatenate([w, jnp.zeros((pad,), jnp.float32)])
    src3 = src.reshape(_NW, _K, _C)
    dst3 = dst.reshape(_NW, _K, _C)
    w2 = wp.reshape(_NW, _K, _C)

    h1 = _matmul(x, W1, 1000)
    p1 = _sc_aggregate(h1, src3, dst3, w2)
    h2 = _combine_relu_mm(p1, W2)
    p2 = _sc_aggregate(h2, src3, dst3, w2)
    out = _combine(p2)
    return out[:n]


# speedup vs baseline: 1.4549x; 1.4549x over previous
import functools

import jax
import jax.numpy as jnp
from jax import lax
from jax.experimental import pallas as pl
from jax.experimental.pallas import tpu as pltpu
from jax.experimental.pallas import tpu_sc as plsc

_NC = 2
_NS = 16
_NW = _NC * _NS
_L = 16
_C = 128
_K = 80
_EPT = _K * _C
_EP = _NW * _EPT
_NG = 4
_NB = 2 * _NG
_RPT = 640
_NPAD = _NS * _RPT


def _mm_body(x_ref, w_ref, o_ref):
    o_ref[...] = jnp.dot(x_ref[...], w_ref[...],
                         preferred_element_type=jnp.float32)


def _matmul(x, w, block_rows):
    n, din = x.shape
    dout = w.shape[1]
    return pl.pallas_call(
        _mm_body,
        grid=(n // block_rows,),
        in_specs=[
            pl.BlockSpec((block_rows, din), lambda i: (i, 0)),
            pl.BlockSpec((din, dout), lambda i: (0, 0)),
        ],
        out_specs=pl.BlockSpec((block_rows, dout), lambda i: (i, 0)),
        out_shape=jax.ShapeDtypeStruct((n, dout), jnp.float32),
    )(x, w)


def _combine_mm_body(p_ref, w_ref, o_ref):
    h = jnp.maximum(p_ref[0] + p_ref[1], 0.0)
    o_ref[...] = jnp.dot(h, w_ref[...], preferred_element_type=jnp.float32)


def _combine_relu_mm(p, w, block_rows=1024):
    n = p.shape[1]
    dout = w.shape[1]
    return pl.pallas_call(
        _combine_mm_body,
        grid=(n // block_rows,),
        in_specs=[
            pl.BlockSpec((2, block_rows, p.shape[2]), lambda i: (0, i, 0)),
            pl.BlockSpec(w.shape, lambda i: (0, 0)),
        ],
        out_specs=pl.BlockSpec((block_rows, dout), lambda i: (i, 0)),
        out_shape=jax.ShapeDtypeStruct((n, dout), jnp.float32),
    )(p, w)


def _combine_body(p_ref, o_ref):
    o_ref[...] = p_ref[0] + p_ref[1]


def _combine(p, block_rows=1024):
    n, d = p.shape[1], p.shape[2]
    return pl.pallas_call(
        _combine_body,
        grid=(n // block_rows,),
        in_specs=[pl.BlockSpec((2, block_rows, d), lambda i: (0, i, 0))],
        out_specs=pl.BlockSpec((block_rows, d), lambda i: (i, 0)),
        out_shape=jax.ShapeDtypeStruct((n, d), jnp.float32),
    )(p)


def _sc_aggregate(h, src3, dst3, w2):
    mesh = plsc.VectorSubcoreMesh(core_axis_name="c", subcore_axis_name="s")

    @functools.partial(
        pl.kernel,
        out_type=jax.ShapeDtypeStruct((_NC, _NPAD, 16), jnp.float32),
        mesh=mesh,
        scratch_types=[
            pltpu.VMEM((_K, _C), jnp.int32),
            pltpu.VMEM((_K, _C), jnp.int32),
            pltpu.VMEM((_K, _C), jnp.float32),
        ] + [pltpu.VMEM((_C, 16), jnp.float32) for _ in range(_NB)] + [
            pltpu.VMEM((_C, 16), jnp.float32),
            pltpu.VMEM_SHARED((_NPAD, 16), jnp.float32),
        ] + [pltpu.SemaphoreType.DMA for _ in range(_NB)],
        compiler_params=pltpu.CompilerParams(use_tc_tiling_on_sc=False),
    )
    def agg(h_hbm, src_hbm, dst_hbm, w_hbm, out_hbm, src_v, dst_v, w_v,
            *bufs):
        rows = bufs[:_NB]
        zbuf = bufs[_NB]
        acc = bufs[_NB + 1]
        sems = bufs[_NB + 2:]
        core = lax.axis_index("c")
        sub = lax.axis_index("s")
        tid = core * _NS + sub

        pltpu.sync_copy(src_hbm.at[tid], src_v)
        pltpu.sync_copy(dst_hbm.at[tid], dst_v)
        pltpu.sync_copy(w_hbm.at[tid], w_v)

        zero = jnp.zeros((_L,), jnp.float32)
        for i in range(_C):
            zbuf[i] = zero
        for k in range(_RPT // _C):
            pltpu.sync_copy(zbuf, acc.at[pl.ds(sub * _RPT + k * _C, _C)])
        plsc.subcore_barrier()

        dnums = lax.GatherDimensionNumbers(
            offset_dims=(), collapsed_slice_dims=(0,), start_index_map=(0,))

        def gath(c, b):
            return pltpu.async_copy(h_hbm.at[src_v.at[c]], rows[b], sems[b])

        def proc(c, b):
            for g in range(_C // _L):
                w16 = w_v[c, pl.ds(g * _L, _L)]
                for e in range(_L):
                    wb = lax.gather(
                        w16, jnp.full((_L, 1), e, jnp.int32), dnums, (1,),
                        mode=lax.GatherScatterMode.PROMISE_IN_BOUNDS)
                    r = g * _L + e
                    rows[b][r] = rows[b][r] * wb
            return pltpu.async_copy(rows[b], acc.at[dst_v.at[c]], sems[b],
                                    add=True)

        primeA = [gath(b, b) for b in range(_NG)]

        def super_body(j, carry):
            base = j * 2 * _NG
            gB = [gath(base + _NG + b, _NG + b) for b in range(_NG)]
            sA = []
            for b in range(_NG):
                pltpu.make_async_copy(
                    h_hbm.at[src_v.at[base + b]], rows[b], sems[b]).wait()
                sA.append(proc(base + b, b))
            for b in range(_NG):
                sA[b].wait()

            @pl.when(j < _K // (2 * _NG) - 1)
            def _():
                for b in range(_NG):
                    gath(base + 2 * _NG + b, b)

            for b in range(_NG):
                gB[b].wait()
            sB = [proc(base + _NG + b, _NG + b) for b in range(_NG)]
            for b in range(_NG):
                sB[b].wait()
            return carry

        lax.fori_loop(0, _K // (2 * _NG), super_body, 0)
        plsc.subcore_barrier()
        pltpu.sync_copy(acc.at[pl.ds(sub * _RPT, _RPT)],
                        out_hbm.at[core, pl.ds(sub * _RPT, _RPT)])

    return agg(h, src3, dst3, w2)


def kernel(x, edge_index, edge_weight, W1, W2):
    n = x.shape[0]
    e = edge_index.shape[1]
    ei = edge_index.astype(jnp.int32)
    w = edge_weight.astype(jnp.float32)
    pad = _EP - e
    pad_iota = jnp.arange(pad, dtype=jnp.int32)
    src = jnp.concatenate([ei[0], pad_iota % n])
    dst = jnp.concatenate([ei[1], pad_iota % _NPAD])
    wp = jnp.concatenate([w, jnp.zeros((pad,), jnp.float32)])
    src3 = src.reshape(_NW, _K, _C)
    dst3 = dst.reshape(_NW, _K, _C)
    w2 = wp.reshape(_NW, _K, _C)

    h1 = _matmul(x, W1, 1000)
    p1 = _sc_aggregate(h1, src3, dst3, w2)
    h2 = _combine_relu_mm(p1, W2)
    p2 = _sc_aggregate(h2, src3, dst3, w2)
    out = _combine(p2)
    return out[:n]


# speedup vs baseline: 1.7953x; 1.2340x over previous
import functools

import jax
import jax.numpy as jnp
from jax import lax
from jax.experimental import pallas as pl
from jax.experimental.pallas import tpu as pltpu
from jax.experimental.pallas import tpu_sc as plsc

_NC = 2
_NS = 16
_NW = _NC * _NS
_L = 16
_C = 128
_K = 80
_EPT = _K * _C
_EP = _NW * _EPT
_NG = 4
_NB = 2 * _NG
_RPT = 640
_NPAD = _NS * _RPT


def _mm_body(x_ref, w_ref, o_ref):
    o_ref[...] = jnp.dot(x_ref[...], w_ref[...],
                         preferred_element_type=jnp.float32)


def _matmul(x, w, block_rows):
    n, din = x.shape
    dout = w.shape[1]
    return pl.pallas_call(
        _mm_body,
        grid=(n // block_rows,),
        in_specs=[
            pl.BlockSpec((block_rows, din), lambda i: (i, 0)),
            pl.BlockSpec((din, dout), lambda i: (0, 0)),
        ],
        out_specs=pl.BlockSpec((block_rows, dout), lambda i: (i, 0)),
        out_shape=jax.ShapeDtypeStruct((n, dout), jnp.float32),
    )(x, w)


def _combine_mm_body(p_ref, w_ref, o_ref):
    h = jnp.maximum(p_ref[0] + p_ref[1], 0.0)
    o_ref[...] = jnp.dot(h, w_ref[...], preferred_element_type=jnp.float32)


def _combine_relu_mm(p128, w128, block_rows=160):
    m = p128.shape[1]
    return pl.pallas_call(
        _combine_mm_body,
        grid=(m // block_rows,),
        in_specs=[
            pl.BlockSpec((2, block_rows, 128), lambda i: (0, i, 0)),
            pl.BlockSpec((128, 128), lambda i: (0, 0)),
        ],
        out_specs=pl.BlockSpec((block_rows, 128), lambda i: (i, 0)),
        out_shape=jax.ShapeDtypeStruct((m, 128), jnp.float32),
    )(p128, w128)


def _combine_body(p_ref, o_ref):
    m = o_ref.shape[0]
    o_ref[...] = p_ref[0, :m] + p_ref[1, :m]


def _combine_packed(p128, out_rows):
    return pl.pallas_call(
        _combine_body,
        grid=(1,),
        in_specs=[pl.BlockSpec(p128.shape, lambda i: (0, 0, 0))],
        out_specs=pl.BlockSpec((out_rows, 128), lambda i: (0, 0)),
        out_shape=jax.ShapeDtypeStruct((out_rows, 128), jnp.float32),
    )(p128)


def _sc_aggregate(h, src3, dst3, w2):
    mesh = plsc.VectorSubcoreMesh(core_axis_name="c", subcore_axis_name="s")

    @functools.partial(
        pl.kernel,
        out_type=jax.ShapeDtypeStruct((_NC, _NPAD, 16), jnp.float32),
        mesh=mesh,
        scratch_types=[
            pltpu.VMEM((_K, _C), jnp.int32),
            pltpu.VMEM((_K, _C), jnp.int32),
            pltpu.VMEM((_K, _C), jnp.float32),
        ] + [pltpu.VMEM((_C, 16), jnp.float32) for _ in range(_NB)] + [
            pltpu.VMEM((_C, 16), jnp.float32),
            pltpu.VMEM_SHARED((_NPAD, 16), jnp.float32),
        ] + [pltpu.SemaphoreType.DMA for _ in range(_NB)],
        compiler_params=pltpu.CompilerParams(use_tc_tiling_on_sc=False),
    )
    def agg(h_hbm, src_hbm, dst_hbm, w_hbm, out_hbm, src_v, dst_v, w_v,
            *bufs):
        rows = bufs[:_NB]
        zbuf = bufs[_NB]
        acc = bufs[_NB + 1]
        sems = bufs[_NB + 2:]
        core = lax.axis_index("c")
        sub = lax.axis_index("s")
        tid = core * _NS + sub

        pltpu.sync_copy(src_hbm.at[tid], src_v)
        pltpu.sync_copy(dst_hbm.at[tid], dst_v)
        pltpu.sync_copy(w_hbm.at[tid], w_v)

        zero = jnp.zeros((_L,), jnp.float32)
        for i in range(_C):
            zbuf[i] = zero
        for k in range(_RPT // _C):
            pltpu.sync_copy(zbuf, acc.at[pl.ds(sub * _RPT + k * _C, _C)])
        plsc.subcore_barrier()

        dnums = lax.GatherDimensionNumbers(
            offset_dims=(), collapsed_slice_dims=(0,), start_index_map=(0,))

        def gath(c, b):
            return pltpu.async_copy(h_hbm.at[src_v.at[c]], rows[b], sems[b])

        def proc(c, b):
            for g in range(_C // _L):
                w16 = w_v[c, pl.ds(g * _L, _L)]
                for e in range(_L):
                    wb = lax.gather(
                        w16, jnp.full((_L, 1), e, jnp.int32), dnums, (1,),
                        mode=lax.GatherScatterMode.PROMISE_IN_BOUNDS)
                    r = g * _L + e
                    rows[b][r] = rows[b][r] * wb
            return pltpu.async_copy(rows[b], acc.at[dst_v.at[c]], sems[b],
                                    add=True)

        primeA = [gath(b, b) for b in range(_NG)]

        def super_body(j, carry):
            base = j * 2 * _NG
            gB = [gath(base + _NG + b, _NG + b) for b in range(_NG)]
            sA = []
            for b in range(_NG):
                pltpu.make_async_copy(
                    h_hbm.at[src_v.at[base + b]], rows[b], sems[b]).wait()
                sA.append(proc(base + b, b))
            for b in range(_NG):
                sA[b].wait()

            @pl.when(j < _K // (2 * _NG) - 1)
            def _():
                for b in range(_NG):
                    gath(base + 2 * _NG + b, b)

            for b in range(_NG):
                gB[b].wait()
            sB = [proc(base + _NG + b, _NG + b) for b in range(_NG)]
            for b in range(_NG):
                sB[b].wait()
            return carry

        lax.fori_loop(0, _K // (2 * _NG), super_body, 0)
        plsc.subcore_barrier()
        pltpu.sync_copy(acc.at[pl.ds(sub * _RPT, _RPT)],
                        out_hbm.at[core, pl.ds(sub * _RPT, _RPT)])

    return agg(h, src3, dst3, w2)


def kernel(x, edge_index, edge_weight, W1, W2):
    n = x.shape[0]
    e = edge_index.shape[1]
    ei = edge_index.astype(jnp.int32)
    w = edge_weight.astype(jnp.float32)
    pad = _EP - e
    pad_iota = jnp.arange(pad, dtype=jnp.int32)
    src = jnp.concatenate([ei[0], pad_iota % n])
    dst = jnp.concatenate([ei[1], pad_iota % _NPAD])
    wp = jnp.concatenate([w, jnp.zeros((pad,), jnp.float32)])
    src3 = src.reshape(_NW, _K, _C)
    dst3 = dst.reshape(_NW, _K, _C)
    w2 = wp.reshape(_NW, _K, _C)

    W2k = jnp.kron(jnp.eye(8, dtype=jnp.float32), W2)

    h1 = _matmul(x, W1, 1000)
    p1 = _sc_aggregate(h1, src3, dst3, w2)
    h2p = _combine_relu_mm(p1.reshape(2, _NPAD // 8, 128), W2k)
    p2 = _sc_aggregate(h2p.reshape(_NPAD, 16), src3, dst3, w2)
    outp = _combine_packed(p2.reshape(2, _NPAD // 8, 128), n // 8)
    return outp.reshape(n, 16)


# speedup vs baseline: 1.8134x; 1.0101x over previous
import functools

import jax
import jax.numpy as jnp
from jax import lax
from jax.experimental import pallas as pl
from jax.experimental.pallas import tpu as pltpu
from jax.experimental.pallas import tpu_sc as plsc

_NC = 2
_NS = 16
_NW = _NC * _NS
_L = 16
_C = 128
_K = 80
_EPT = _K * _C
_EP = _NW * _EPT
_NG = 4
_NB = 2 * _NG
_RPT = 640
_NPAD = _NS * _RPT


def _mm_body(x_ref, w_ref, o_ref):
    o_ref[...] = jnp.dot(x_ref[...], w_ref[...],
                         preferred_element_type=jnp.float32)


def _matmul(x, w, block_rows):
    n, din = x.shape
    dout = w.shape[1]
    return pl.pallas_call(
        _mm_body,
        grid=(n // block_rows,),
        in_specs=[
            pl.BlockSpec((block_rows, din), lambda i: (i, 0)),
            pl.BlockSpec((din, dout), lambda i: (0, 0)),
        ],
        out_specs=pl.BlockSpec((block_rows, dout), lambda i: (i, 0)),
        out_shape=jax.ShapeDtypeStruct((n, dout), jnp.float32),
    )(x, w)


def _combine_mm_body(p_ref, w_ref, o_ref):
    h = jnp.maximum(p_ref[0] + p_ref[1], 0.0)
    o_ref[...] = jnp.dot(h, w_ref[...], preferred_element_type=jnp.float32)


def _combine_relu_mm(p128, w128, block_rows=160):
    m = p128.shape[1]
    return pl.pallas_call(
        _combine_mm_body,
        grid=(m // block_rows,),
        in_specs=[
            pl.BlockSpec((2, block_rows, 128), lambda i: (0, i, 0)),
            pl.BlockSpec((128, 128), lambda i: (0, 0)),
        ],
        out_specs=pl.BlockSpec((block_rows, 128), lambda i: (i, 0)),
        out_shape=jax.ShapeDtypeStruct((m, 128), jnp.float32),
    )(p128, w128)


def _combine_body(p_ref, o_ref):
    m = o_ref.shape[0]
    o_ref[...] = p_ref[0, :m] + p_ref[1, :m]


def _combine_packed(p128, out_rows):
    return pl.pallas_call(
        _combine_body,
        grid=(1,),
        in_specs=[pl.BlockSpec(p128.shape, lambda i: (0, 0, 0))],
        out_specs=pl.BlockSpec((out_rows, 128), lambda i: (0, 0)),
        out_shape=jax.ShapeDtypeStruct((out_rows, 128), jnp.float32),
    )(p128)


def _sc_aggregate(h, src3, dst3, w2):
    mesh = plsc.VectorSubcoreMesh(core_axis_name="c", subcore_axis_name="s")

    @functools.partial(
        pl.kernel,
        out_type=jax.ShapeDtypeStruct((_NC, _NPAD, 16), jnp.float32),
        mesh=mesh,
        scratch_types=[
            pltpu.VMEM((_K, _C), jnp.int32),
            pltpu.VMEM((_K, _C), jnp.int32),
            pltpu.VMEM((_K, _C), jnp.float32),
        ] + [pltpu.VMEM((_C, 16), jnp.float32) for _ in range(_NB)] + [
            pltpu.VMEM((_C, 16), jnp.float32),
            pltpu.VMEM_SHARED((_NPAD, 16), jnp.float32),
        ] + [pltpu.SemaphoreType.DMA for _ in range(_NB)],
        compiler_params=pltpu.CompilerParams(use_tc_tiling_on_sc=False),
    )
    def agg(h_hbm, src_hbm, dst_hbm, w_hbm, out_hbm, src_v, dst_v, w_v,
            *bufs):
        rows = bufs[:_NB]
        zbuf = bufs[_NB]
        acc = bufs[_NB + 1]
        sems = bufs[_NB + 2:]
        core = lax.axis_index("c")
        sub = lax.axis_index("s")
        tid = core * _NS + sub

        pltpu.sync_copy(src_hbm.at[tid], src_v)
        pltpu.sync_copy(dst_hbm.at[tid], dst_v)
        pltpu.sync_copy(w_hbm.at[tid], w_v)

        zero = jnp.zeros((_L,), jnp.float32)
        for i in range(_C):
            zbuf[i] = zero
        for k in range(_RPT // _C):
            pltpu.sync_copy(zbuf, acc.at[pl.ds(sub * _RPT + k * _C, _C)])
        plsc.subcore_barrier()

        dnums = lax.GatherDimensionNumbers(
            offset_dims=(), collapsed_slice_dims=(0,), start_index_map=(0,))

        def gath(c, b):
            return pltpu.async_copy(h_hbm.at[src_v.at[c]], rows[b], sems[b])

        def proc(c, b):
            for g in range(_C // _L):
                w16 = w_v[c, pl.ds(g * _L, _L)]
                for e in range(_L):
                    wb = lax.gather(
                        w16, jnp.full((_L, 1), e, jnp.int32), dnums, (1,),
                        mode=lax.GatherScatterMode.PROMISE_IN_BOUNDS)
                    r = g * _L + e
                    rows[b][r] = rows[b][r] * wb
            return pltpu.async_copy(rows[b], acc.at[dst_v.at[c]], sems[b],
                                    add=True)

        primeA = [gath(b, b) for b in range(_NG)]

        def super_body(j, carry):
            base = j * 2 * _NG
            gB = [gath(base + _NG + b, _NG + b) for b in range(_NG)]
            sA = []
            for b in range(_NG):
                pltpu.make_async_copy(
                    h_hbm.at[src_v.at[base + b]], rows[b], sems[b]).wait()
                sA.append(proc(base + b, b))
            for b in range(_NG):
                sA[b].wait()

            @pl.when(j < _K // (2 * _NG) - 1)
            def _():
                for b in range(_NG):
                    gath(base + 2 * _NG + b, b)

            for b in range(_NG):
                gB[b].wait()
            sB = [proc(base + _NG + b, _NG + b) for b in range(_NG)]
            for b in range(_NG):
                sB[b].wait()
            return carry

        lax.fori_loop(0, _K // (2 * _NG), super_body, 0)
        plsc.subcore_barrier()
        pltpu.sync_copy(acc.at[pl.ds(sub * _RPT, _RPT)],
                        out_hbm.at[core, pl.ds(sub * _RPT, _RPT)])

    return agg(h, src3, dst3, w2)


def kernel(x, edge_index, edge_weight, W1, W2):
    n = x.shape[0]
    e = edge_index.shape[1]
    ei = edge_index.astype(jnp.int32)
    w = edge_weight.astype(jnp.float32)
    pad = _EP - e
    pad_iota = jnp.arange(pad, dtype=jnp.int32).reshape(pad // _C, _C)
    src = jnp.concatenate([ei[0].reshape(e // _C, _C), pad_iota % n])
    dst = jnp.concatenate([ei[1].reshape(e // _C, _C), pad_iota % _NPAD])
    wp = jnp.concatenate([w.reshape(e // _C, _C),
                          jnp.zeros((pad // _C, _C), jnp.float32)])
    src3 = src.reshape(_NW, _K, _C)
    dst3 = dst.reshape(_NW, _K, _C)
    w2 = wp.reshape(_NW, _K, _C)

    W2k = jnp.kron(jnp.eye(8, dtype=jnp.float32), W2)

    h1 = _matmul(x, W1, 2000)
    p1 = _sc_aggregate(h1, src3, dst3, w2)
    h2p = _combine_relu_mm(p1.reshape(2, _NPAD // 8, 128), W2k)
    p2 = _sc_aggregate(h2p.reshape(_NPAD, 16), src3, dst3, w2)
    outp = _combine_packed(p2.reshape(2, _NPAD // 8, 128), n // 8)
    return outp.reshape(n, 16)
